# baseline (speedup 1.0000x reference)
import jax
import jax.numpy as jnp
from jax import lax
from jax.experimental import pallas as pl
from jax.experimental.pallas import tpu as pltpu
from jax.experimental.pallas import tpu_sc as plsc

N = 10000
E = 320000
D = 128

NC = 2
NS = 16
SB = 50
IB = 8
RING = 4
LOOK = 3
EPT = E // (NC * NS)
IRPT = EPT // SB
NIT = IRPT // IB
NPAD = 10240
RPT = NPAD // NS
RCO = 64


_BM = 2000


def _mm_body(h_ref, w_ref, n_ref, o_ref):
    o_ref[...] = jnp.dot(h_ref[...], w_ref[...],
                         preferred_element_type=jnp.float32) * n_ref[...]


def _matmul_scale(h, W, norm):
    return pl.pallas_call(
        _mm_body,
        grid=(N // _BM,),
        in_specs=[
            pl.BlockSpec((_BM, D), lambda i: (i, 0)),
            pl.BlockSpec((D, D), lambda i: (0, 0)),
            pl.BlockSpec((_BM, 1), lambda i: (i, 0)),
        ],
        out_specs=pl.BlockSpec((_BM, D), lambda i: (i, 0)),
        out_shape=jax.ShapeDtypeStruct((N, D), jnp.float32),
    )(h, W, norm)


def _merge_body(p_ref, n_ref, o_ref):
    o_ref[...] = (p_ref[0] + p_ref[1]) * n_ref[...]


def _merge_scale(parts, norm):
    return pl.pallas_call(
        _merge_body,
        grid=(N // _BM,),
        in_specs=[
            pl.BlockSpec((NC, _BM, D), lambda i: (0, i, 0)),
            pl.BlockSpec((_BM, 1), lambda i: (i, 0)),
        ],
        out_specs=pl.BlockSpec((_BM, D), lambda i: (i, 0)),
        out_shape=jax.ShapeDtypeStruct((N, D), jnp.float32),
    )(parts, norm)


def _sc_body(hw, srcm, dstm, zer, out,
             agg_sh, idx_s, idx_d, rows, co, gsem, ssem, isem):
    c = lax.axis_index("c")
    s = lax.axis_index("s")
    row0 = s * RPT

    pltpu.sync_copy(zer, agg_sh.at[pl.ds(row0, RPT)])
    erow0 = (c * NS + s) * IRPT

    pltpu.sync_copy(srcm.at[pl.ds(erow0, IB)], idx_s.at[0])
    pltpu.sync_copy(dstm.at[pl.ds(erow0, IB)], idx_d.at[0])
    pltpu.async_copy(srcm.at[pl.ds(erow0 + IB, IB)], idx_s.at[1], isem)
    pltpu.async_copy(dstm.at[pl.ds(erow0 + IB, IB)], idx_d.at[1], isem)
    plsc.subcore_barrier()

    for g in range(LOOK):
        pltpu.async_copy(hw.at[idx_s.at[0, g]], rows.at[g], gsem.at[g])

    def chunk(i, carry):
        q = lax.rem(i, 2)
        qn = lax.rem(i + 1, 2)
        for jj in range(IB):
            p = jj % RING
            pn = (jj + LOOK) % RING
            j = i * IB + jj
            pltpu.make_async_copy(hw.at[idx_s.at[q, jj]],
                                  rows.at[p], gsem.at[p]).wait()
            pltpu.async_copy(rows.at[p], agg_sh.at[idx_d.at[q, jj]],
                             ssem.at[p], add=True)
            if jj == 0:
                @pl.when(j >= 1)
                def _():
                    pltpu.make_async_copy(rows.at[pn],
                                          agg_sh.at[idx_d.at[qn, IB - 1]],
                                          ssem.at[pn]).wait()
                @pl.when((i >= 1) & (i + 1 < NIT))
                def _():
                    pltpu.async_copy(srcm.at[pl.ds(erow0 + (i + 1) * IB, IB)],
                                     idx_s.at[qn], isem)
                    pltpu.async_copy(dstm.at[pl.ds(erow0 + (i + 1) * IB, IB)],
                                     idx_d.at[qn], isem)
            else:
                pltpu.make_async_copy(rows.at[pn],
                                      agg_sh.at[idx_d.at[q, jj - 1]],
                                      ssem.at[pn]).wait()
            if jj == IB - LOOK - 1:
                @pl.when(i + 1 < NIT)
                def _():
                    pltpu.make_async_copy(
                        srcm.at[pl.ds(erow0 + (i + 1) * IB, IB)],
                        idx_s.at[qn], isem).wait()
                    pltpu.make_async_copy(
                        dstm.at[pl.ds(erow0 + (i + 1) * IB, IB)],
                        idx_d.at[qn], isem).wait()
            if jj < IB - LOOK:
                pltpu.async_copy(hw.at[idx_s.at[q, jj + LOOK]],
                                 rows.at[pn], gsem.at[pn])
            else:
                @pl.when(i + 1 < NIT)
                def _():
                    pltpu.async_copy(hw.at[idx_s.at[qn, jj + LOOK - IB]],
                                     rows.at[pn], gsem.at[pn])
        return carry

    lax.fori_loop(0, NIT, chunk, 0)

    pltpu.make_async_copy(rows.at[(IRPT - 1) % RING],
                          agg_sh.at[idx_d.at[(NIT - 1) % 2, IB - 1]],
                          ssem.at[(IRPT - 1) % RING]).wait()
    plsc.subcore_barrier()

    for k in range(RPT // RCO):
        r0 = row0 + k * RCO
        pltpu.sync_copy(agg_sh.at[pl.ds(r0, RCO)], co)
        pltpu.sync_copy(co, out.at[c, pl.ds(r0, RCO)])


_sc_agg = pl.kernel(
    _sc_body,
    out_type=jax.ShapeDtypeStruct((NC, NPAD, D), jnp.float32),
    mesh=plsc.VectorSubcoreMesh(core_axis_name="c", subcore_axis_name="s"),
    scratch_types=[
        pltpu.MemorySpace.VMEM_SHARED((NPAD, D), jnp.float32),
        pltpu.MemorySpace.VMEM((2, IB, SB), jnp.int32),
        pltpu.MemorySpace.VMEM((2, IB, SB), jnp.int32),
        pltpu.MemorySpace.VMEM((RING, SB, D), jnp.float32),
        pltpu.MemorySpace.VMEM((RCO, D), jnp.float32),
        pltpu.SemaphoreType.DMA((RING,)),
        pltpu.SemaphoreType.DMA((RING,)),
        pltpu.SemaphoreType.DMA,
    ],
)


def kernel(h, edge_index, norm, W):
    hw = _matmul_scale(h, W, norm)
    srcm = edge_index[0].reshape(E // SB, SB)
    dstm = edge_index[1].reshape(E // SB, SB)
    zer = jnp.zeros((RPT, D), jnp.float32)
    parts = _sc_agg(hw, srcm, dstm, zer)
    return _merge_scale(parts, norm)


# speedup vs baseline: 1.0395x; 1.0395x over previous
import jax
import jax.numpy as jnp
from jax import lax
from jax.experimental import pallas as pl
from jax.experimental.pallas import tpu as pltpu
from jax.experimental.pallas import tpu_sc as plsc

N = 10000
E = 320000
D = 128

NC = 2
NS = 16
SB = 125
IB = 8
EPT = E // (NC * NS)
IRPT = EPT // SB
NIT = IRPT // IB
NPAD = 10240
RPT = NPAD // NS
RCO = 128


_BM = 2000


def _mm_body(h_ref, w_ref, n_ref, o_ref):
    o_ref[...] = jnp.dot(h_ref[...], w_ref[...],
                         preferred_element_type=jnp.float32) * n_ref[...]


def _matmul_scale(h, W, norm):
    return pl.pallas_call(
        _mm_body,
        grid=(N // _BM,),
        in_specs=[
            pl.BlockSpec((_BM, D), lambda i: (i, 0)),
            pl.BlockSpec((D, D), lambda i: (0, 0)),
            pl.BlockSpec((_BM, 1), lambda i: (i, 0)),
        ],
        out_specs=pl.BlockSpec((_BM, D), lambda i: (i, 0)),
        out_shape=jax.ShapeDtypeStruct((N, D), jnp.float32),
    )(h, W, norm)


def _merge_body(p_ref, n_ref, o_ref):
    o_ref[...] = (p_ref[0] + p_ref[1]) * n_ref[...]


def _merge_scale(parts, norm):
    return pl.pallas_call(
        _merge_body,
        grid=(N // _BM,),
        in_specs=[
            pl.BlockSpec((NC, _BM, D), lambda i: (0, i, 0)),
            pl.BlockSpec((_BM, 1), lambda i: (i, 0)),
        ],
        out_specs=pl.BlockSpec((_BM, D), lambda i: (i, 0)),
        out_shape=jax.ShapeDtypeStruct((N, D), jnp.float32),
    )(parts, norm)


def _sc_body(hw, srcm, dstm, zer, out, agg_sh, idx_s, idx_d, rows, gsem, isem):
    c = lax.axis_index("c")
    s = lax.axis_index("s")
    row0 = s * RPT

    erow0 = (c * NS + s) * IRPT

    pltpu.sync_copy(srcm.at[pl.ds(erow0, IB)], idx_s.at[0])
    pltpu.sync_copy(dstm.at[pl.ds(erow0, IB)], idx_d.at[0])
    pltpu.async_copy(srcm.at[pl.ds(erow0 + IB, IB)], idx_s.at[1], isem)
    pltpu.async_copy(dstm.at[pl.ds(erow0 + IB, IB)], idx_d.at[1], isem)

    pltpu.async_copy(hw.at[idx_s.at[0, 0]], rows.at[0, pl.ds(0, SB)], gsem.at[0])
    pltpu.async_copy(hw.at[idx_s.at[0, 1]], rows.at[1, pl.ds(0, SB)], gsem.at[1])

    pltpu.sync_copy(zer, agg_sh.at[pl.ds(row0, RPT)])
    plsc.subcore_barrier()

    def chunk(i, carry):
        q = lax.rem(i, 2)
        qn = lax.rem(i + 1, 2)
        for j in range(IB):
            p = j % 2
            pltpu.make_async_copy(hw.at[idx_s.at[q, j]],
                                  rows.at[p, pl.ds(0, SB)], gsem.at[p]).wait()
            pltpu.sync_copy(rows.at[p, pl.ds(0, SB)],
                            agg_sh.at[idx_d.at[q, j]], add=True)
            if j == IB - 3:
                @pl.when(i + 1 < NIT)
                def _():
                    pltpu.make_async_copy(
                        srcm.at[pl.ds(erow0 + (i + 1) * IB, IB)],
                        idx_s.at[qn], isem).wait()
                    pltpu.make_async_copy(
                        dstm.at[pl.ds(erow0 + (i + 1) * IB, IB)],
                        idx_d.at[qn], isem).wait()
            if j < IB - 2:
                pltpu.async_copy(hw.at[idx_s.at[q, j + 2]],
                                 rows.at[p, pl.ds(0, SB)], gsem.at[p])
            else:
                @pl.when(i + 1 < NIT)
                def _():
                    pltpu.async_copy(hw.at[idx_s.at[qn, j + 2 - IB]],
                                     rows.at[p, pl.ds(0, SB)], gsem.at[p])
            if j == IB - 1:
                @pl.when(i + 2 < NIT)
                def _():
                    pltpu.async_copy(srcm.at[pl.ds(erow0 + (i + 2) * IB, IB)],
                                     idx_s.at[q], isem)
                    pltpu.async_copy(dstm.at[pl.ds(erow0 + (i + 2) * IB, IB)],
                                     idx_d.at[q], isem)
        return carry

    lax.fori_loop(0, NIT, chunk, 0)
    plsc.subcore_barrier()

    for k in range(RPT // RCO):
        r0 = row0 + k * RCO
        pltpu.sync_copy(agg_sh.at[pl.ds(r0, RCO)], out.at[c, pl.ds(r0, RCO)])


_sc_agg = pl.kernel(
    _sc_body,
    out_type=jax.ShapeDtypeStruct((NC, NPAD, D), jnp.float32),
    mesh=plsc.VectorSubcoreMesh(core_axis_name="c", subcore_axis_name="s"),
    scratch_types=[
        pltpu.MemorySpace.VMEM_SHARED((NPAD, D), jnp.float32),
        pltpu.MemorySpace.VMEM((2, IB, SB), jnp.int32),
        pltpu.MemorySpace.VMEM((2, IB, SB), jnp.int32),
        pltpu.MemorySpace.VMEM((2, RCO, D), jnp.float32),
        pltpu.SemaphoreType.DMA((2,)),
        pltpu.SemaphoreType.DMA,
    ],
)


def kernel(h, edge_index, norm, W):
    hw = _matmul_scale(h, W, norm)
    srcm = edge_index[0].reshape(E // SB, SB)
    dstm = edge_index[1].reshape(E // SB, SB)
    zer = jnp.zeros((RPT, D), jnp.float32)
    parts = _sc_agg(hw, srcm, dstm, zer)
    return _merge_scale(parts, norm)


# speedup vs baseline: 1.0415x; 1.0019x over previous
import jax
import jax.numpy as jnp
from jax import lax
from jax.experimental import pallas as pl
from jax.experimental.pallas import tpu as pltpu
from jax.experimental.pallas import tpu_sc as plsc

N = 10000
E = 320000
D = 128

NC = 2
NS = 16
SB = 125
IB = 8
EPT = E // (NC * NS)
IRPT = EPT // SB
NIT = IRPT // IB
NPAD = 10240
RPT = NPAD // NS
RCO = 128


_BM = 2000


def _mm_body(h_ref, w_ref, n_ref, o_ref):
    o_ref[...] = jnp.dot(h_ref[...], w_ref[...],
                         preferred_element_type=jnp.float32) * n_ref[...]


def _matmul_scale(h, W, norm):
    return pl.pallas_call(
        _mm_body,
        grid=(N // _BM,),
        in_specs=[
            pl.BlockSpec((_BM, D), lambda i: (i, 0)),
            pl.BlockSpec((D, D), lambda i: (0, 0)),
            pl.BlockSpec((_BM, 1), lambda i: (i, 0)),
        ],
        out_specs=pl.BlockSpec((_BM, D), lambda i: (i, 0)),
        out_shape=jax.ShapeDtypeStruct((N, D), jnp.float32),
    )(h, W, norm)


def _merge_body(p_ref, n_ref, o_ref):
    o_ref[...] = (p_ref[0] + p_ref[1]) * n_ref[...]


def _merge_scale(parts, norm):
    return pl.pallas_call(
        _merge_body,
        grid=(N // _BM,),
        in_specs=[
            pl.BlockSpec((NC, _BM, D), lambda i: (0, i, 0)),
            pl.BlockSpec((_BM, 1), lambda i: (i, 0)),
        ],
        out_specs=pl.BlockSpec((_BM, D), lambda i: (i, 0)),
        out_shape=jax.ShapeDtypeStruct((N, D), jnp.float32),
    )(parts, norm)


def _sc_body(hw, srcm, dstm, zer, out, agg_sh, idx_s, idx_d, rows, gsem, isem):
    c = lax.axis_index("c")
    s = lax.axis_index("s")
    row0 = s * RPT

    erow0 = (c * NS + s) * IRPT

    pltpu.async_copy(srcm.at[pl.ds(erow0, IB)], idx_s.at[0], gsem.at[0])
    pltpu.async_copy(dstm.at[pl.ds(erow0, IB)], idx_d.at[0], gsem.at[1])
    pltpu.async_copy(srcm.at[pl.ds(erow0 + IB, IB)], idx_s.at[1], isem)
    pltpu.async_copy(dstm.at[pl.ds(erow0 + IB, IB)], idx_d.at[1], isem)
    pltpu.make_async_copy(srcm.at[pl.ds(erow0, IB)], idx_s.at[0], gsem.at[0]).wait()
    pltpu.make_async_copy(dstm.at[pl.ds(erow0, IB)], idx_d.at[0], gsem.at[1]).wait()

    pltpu.async_copy(hw.at[idx_s.at[0, 0]], rows.at[0, pl.ds(0, SB)], gsem.at[0])
    pltpu.async_copy(hw.at[idx_s.at[0, 1]], rows.at[1, pl.ds(0, SB)], gsem.at[1])

    pltpu.sync_copy(zer, agg_sh.at[pl.ds(row0, RPT)])
    plsc.subcore_barrier()

    def chunk(i, carry):
        q = lax.rem(i, 2)
        qn = lax.rem(i + 1, 2)
        for j in range(IB):
            p = j % 2
            pltpu.make_async_copy(hw.at[idx_s.at[q, j]],
                                  rows.at[p, pl.ds(0, SB)], gsem.at[p]).wait()
            pltpu.sync_copy(rows.at[p, pl.ds(0, SB)],
                            agg_sh.at[idx_d.at[q, j]], add=True)
            if j == IB - 3:
                @pl.when(i + 1 < NIT)
                def _():
                    pltpu.make_async_copy(
                        srcm.at[pl.ds(erow0 + (i + 1) * IB, IB)],
                        idx_s.at[qn], isem).wait()
                    pltpu.make_async_copy(
                        dstm.at[pl.ds(erow0 + (i + 1) * IB, IB)],
                        idx_d.at[qn], isem).wait()
            if j < IB - 2:
                pltpu.async_copy(hw.at[idx_s.at[q, j + 2]],
                                 rows.at[p, pl.ds(0, SB)], gsem.at[p])
            else:
                @pl.when(i + 1 < NIT)
                def _():
                    pltpu.async_copy(hw.at[idx_s.at[qn, j + 2 - IB]],
                                     rows.at[p, pl.ds(0, SB)], gsem.at[p])
            if j == IB - 1:
                @pl.when(i + 2 < NIT)
                def _():
                    pltpu.async_copy(srcm.at[pl.ds(erow0 + (i + 2) * IB, IB)],
                                     idx_s.at[q], isem)
                    pltpu.async_copy(dstm.at[pl.ds(erow0 + (i + 2) * IB, IB)],
                                     idx_d.at[q], isem)
        return carry

    lax.fori_loop(0, NIT, chunk, 0)
    plsc.subcore_barrier()

    for k in range(RPT // RCO):
        r0 = row0 + k * RCO
        pltpu.sync_copy(agg_sh.at[pl.ds(r0, RCO)], out.at[c, pl.ds(r0, RCO)])


_sc_agg = pl.kernel(
    _sc_body,
    out_type=jax.ShapeDtypeStruct((NC, NPAD, D), jnp.float32),
    mesh=plsc.VectorSubcoreMesh(core_axis_name="c", subcore_axis_name="s"),
    scratch_types=[
        pltpu.MemorySpace.VMEM_SHARED((NPAD, D), jnp.float32),
        pltpu.MemorySpace.VMEM((2, IB, SB), jnp.int32),
        pltpu.MemorySpace.VMEM((2, IB, SB), jnp.int32),
        pltpu.MemorySpace.VMEM((2, RCO, D), jnp.float32),
        pltpu.SemaphoreType.DMA((2,)),
        pltpu.SemaphoreType.DMA,
    ],
)


def kernel(h, edge_index, norm, W):
    hw = _matmul_scale(h, W, norm)
    srcm = edge_index[0].reshape(E // SB, SB)
    dstm = edge_index[1].reshape(E // SB, SB)
    zer = jnp.zeros((RPT, D), jnp.float32)
    parts = _sc_agg(hw, srcm, dstm, zer)
    return _merge_scale(parts, norm)


# speedup vs baseline: 1.0428x; 1.0012x over previous
import jax
import jax.numpy as jnp
from jax import lax
from jax.experimental import pallas as pl
from jax.experimental.pallas import tpu as pltpu
from jax.experimental.pallas import tpu_sc as plsc

N = 10000
E = 320000
D = 128

NC = 2
NS = 16
SB = 125
IB = 16
EPT = E // (NC * NS)
IRPT = EPT // SB
NIT = IRPT // IB
NPAD = 10240
RPT = NPAD // NS
RCO = 128


_BM = 2000


def _mm_body(h_ref, w_ref, n_ref, o_ref):
    o_ref[...] = jnp.dot(h_ref[...], w_ref[...],
                         preferred_element_type=jnp.float32) * n_ref[...]


def _matmul_scale(h, W, norm):
    return pl.pallas_call(
        _mm_body,
        grid=(N // _BM,),
        in_specs=[
            pl.BlockSpec((_BM, D), lambda i: (i, 0)),
            pl.BlockSpec((D, D), lambda i: (0, 0)),
            pl.BlockSpec((_BM, 1), lambda i: (i, 0)),
        ],
        out_specs=pl.BlockSpec((_BM, D), lambda i: (i, 0)),
        out_shape=jax.ShapeDtypeStruct((N, D), jnp.float32),
    )(h, W, norm)


def _merge_body(p_ref, n_ref, o_ref):
    o_ref[...] = (p_ref[0] + p_ref[1]) * n_ref[...]


def _merge_scale(parts, norm):
    return pl.pallas_call(
        _merge_body,
        grid=(N // _BM,),
        in_specs=[
            pl.BlockSpec((NC, _BM, D), lambda i: (0, i, 0)),
            pl.BlockSpec((_BM, 1), lambda i: (i, 0)),
        ],
        out_specs=pl.BlockSpec((_BM, D), lambda i: (i, 0)),
        out_shape=jax.ShapeDtypeStruct((N, D), jnp.float32),
    )(parts, norm)


def _sc_body(hw, srcm, dstm, zer, out, agg_sh, idx_s, idx_d, rows, gsem, isem):
    c = lax.axis_index("c")
    s = lax.axis_index("s")
    row0 = s * RPT

    erow0 = (c * NS + s) * IRPT

    pltpu.async_copy(srcm.at[pl.ds(erow0, IB)], idx_s.at[0], gsem.at[0])
    pltpu.async_copy(dstm.at[pl.ds(erow0, IB)], idx_d.at[0], gsem.at[1])
    pltpu.async_copy(srcm.at[pl.ds(erow0 + IB, IB)], idx_s.at[1], isem)
    pltpu.async_copy(dstm.at[pl.ds(erow0 + IB, IB)], idx_d.at[1], isem)
    pltpu.make_async_copy(srcm.at[pl.ds(erow0, IB)], idx_s.at[0], gsem.at[0]).wait()
    pltpu.make_async_copy(dstm.at[pl.ds(erow0, IB)], idx_d.at[0], gsem.at[1]).wait()

    pltpu.async_copy(hw.at[idx_s.at[0, 0]], rows.at[0, pl.ds(0, SB)], gsem.at[0])
    pltpu.async_copy(hw.at[idx_s.at[0, 1]], rows.at[1, pl.ds(0, SB)], gsem.at[1])

    pltpu.sync_copy(zer, agg_sh.at[pl.ds(row0, RPT)])
    plsc.subcore_barrier()

    def chunk(i, carry):
        q = lax.rem(i, 2)
        qn = lax.rem(i + 1, 2)
        for j in range(IB):
            p = j % 2
            pltpu.make_async_copy(hw.at[idx_s.at[q, j]],
                                  rows.at[p, pl.ds(0, SB)], gsem.at[p]).wait()
            pltpu.sync_copy(rows.at[p, pl.ds(0, SB)],
                            agg_sh.at[idx_d.at[q, j]], add=True)
            if j == IB - 3:
                @pl.when(i + 1 < NIT)
                def _():
                    pltpu.make_async_copy(
                        srcm.at[pl.ds(erow0 + (i + 1) * IB, IB)],
                        idx_s.at[qn], isem).wait()
                    pltpu.make_async_copy(
                        dstm.at[pl.ds(erow0 + (i + 1) * IB, IB)],
                        idx_d.at[qn], isem).wait()
            if j < IB - 2:
                pltpu.async_copy(hw.at[idx_s.at[q, j + 2]],
                                 rows.at[p, pl.ds(0, SB)], gsem.at[p])
            else:
                @pl.when(i + 1 < NIT)
                def _():
                    pltpu.async_copy(hw.at[idx_s.at[qn, j + 2 - IB]],
                                     rows.at[p, pl.ds(0, SB)], gsem.at[p])
            if j == IB - 1:
                @pl.when(i + 2 < NIT)
                def _():
                    pltpu.async_copy(srcm.at[pl.ds(erow0 + (i + 2) * IB, IB)],
                                     idx_s.at[q], isem)
                    pltpu.async_copy(dstm.at[pl.ds(erow0 + (i + 2) * IB, IB)],
                                     idx_d.at[q], isem)
        return carry

    lax.fori_loop(0, NIT, chunk, 0)
    plsc.subcore_barrier()

    for k in range(RPT // RCO):
        r0 = row0 + k * RCO
        pltpu.sync_copy(agg_sh.at[pl.ds(r0, RCO)], out.at[c, pl.ds(r0, RCO)])


_sc_agg = pl.kernel(
    _sc_body,
    out_type=jax.ShapeDtypeStruct((NC, NPAD, D), jnp.float32),
    mesh=plsc.VectorSubcoreMesh(core_axis_name="c", subcore_axis_name="s"),
    scratch_types=[
        pltpu.MemorySpace.VMEM_SHARED((NPAD, D), jnp.float32),
        pltpu.MemorySpace.VMEM((2, IB, SB), jnp.int32),
        pltpu.MemorySpace.VMEM((2, IB, SB), jnp.int32),
        pltpu.MemorySpace.VMEM((2, RCO, D), jnp.float32),
        pltpu.SemaphoreType.DMA((2,)),
        pltpu.SemaphoreType.DMA,
    ],
)


def kernel(h, edge_index, norm, W):
    hw = _matmul_scale(h, W, norm)
    srcm = edge_index[0].reshape(E // SB, SB)
    dstm = edge_index[1].reshape(E // SB, SB)
    zer = jnp.zeros((RPT, D), jnp.float32)
    parts = _sc_agg(hw, srcm, dstm, zer)
    return _merge_scale(parts, norm)
